# baseline (speedup 1.0000x reference)
import jax
import jax.numpy as jnp
from jax.experimental import pallas as pl
from jax.experimental.pallas import tpu as pltpu

_B = 64
_G = 13
_GG = _G * _G
_A = 5
_C = 20
_CH = _A * (_C + 5)
_R = 8
_L = _B * _GG // _R
_ANCHORS = ((1.3221, 1.73145), (3.19275, 4.00944), (5.05587, 8.09892),
            (9.47112, 4.84053), (11.2364, 10.0071))
_OBJECT_SCALE = 5.0
_NOOBJECT_SCALE = 1.0
_CLASS_SCALE = 1.0
_COORD_SCALE = 1.0
_THRESH = 0.6
_ONE_T = 1.0 + _THRESH
_EPS = 1e-16


def _body(xt_ref, xd_ref, tgt_ref, tgtT_ref, ts_ref, gate_ref, out_ref):
    f32 = jnp.float32

    tgt = tgt_ref[:, :]
    cls_t = tgt[:, 1:2]
    cx = tgt[:, 2:3] * _G
    cy = tgt[:, 3:4] * _G
    twg = tgt[:, 4:5] * _G
    thg = tgt[:, 5:6] * _G
    gxf = jnp.floor(cx)
    gyf = jnp.floor(cy)
    pcol = (gyf * _G + gxf).astype(jnp.int32)

    best = jnp.full((_B, 1), -1.0, f32)
    acol = jnp.zeros((_B, 1), jnp.int32)
    awb = jnp.full((_B, 1), _ANCHORS[0][0], f32)
    ahb = jnp.full((_B, 1), _ANCHORS[0][1], f32)
    for a, (aw, ah) in enumerate(_ANCHORS):
        inter = jnp.minimum(aw, twg) * jnp.minimum(ah, thg)
        union = aw * ah + twg * thg - inter
        r = inter / (union + _EPS)
        upd = r > best
        best = jnp.where(upd, r, best)
        acol = jnp.where(upd, a, acol)
        awb = jnp.where(upd, aw, awb)
        ahb = jnp.where(upd, ah, ahb)

    gi = jax.lax.broadcasted_iota(jnp.int32, (1, _B, _GG), 2)
    msk = (gi == pcol.reshape(1, _B, 1)).astype(f32)
    w = jnp.sum(xt_ref[:, :, :] * msk, axis=2)
    wt = w.T

    chi = jax.lax.broadcasted_iota(jnp.int32, (_B, _CH), 1)
    base = acol * (_C + 5)
    sel = []
    for c in range(_C + 5):
        m = (chi == base + c).astype(f32)
        sel.append(jnp.sum(wt * m, axis=1, keepdims=True))

    txs, tys, tws, ths, cfs = sel[0], sel[1], sel[2], sel[3], sel[4]

    acc_cls = jnp.float32(0.0)
    for j in range(_C):
        pj = jax.nn.sigmoid(sel[5 + j])
        oh = (cls_t == float(j)).astype(f32)
        acc_cls = acc_cls + jnp.sum((pj - oh) ** 2)
    loss_cls = acc_cls / (_B * _C) * _CLASS_SCALE

    sx_t = jax.nn.sigmoid(txs)
    sy_t = jax.nn.sigmoid(tys)
    pcf_t = jax.nn.sigmoid(cfs)
    px = sx_t + gxf
    py = sy_t + gyf
    pw = jnp.exp(tws) * awb
    ph = jnp.exp(ths) * ahb

    ix1 = jnp.maximum(px - pw * 0.5, cx - twg * 0.5)
    ix2 = jnp.minimum(px + pw * 0.5, cx + twg * 0.5)
    iy1 = jnp.maximum(py - ph * 0.5, cy - thg * 0.5)
    iy2 = jnp.minimum(py + ph * 0.5, cy + thg * 0.5)
    iw = jnp.clip(ix2 - ix1, 0.0, None)
    ih = jnp.clip(iy2 - iy1, 0.0, None)
    inter_t = iw * ih
    iou_t = inter_t / (pw * ph + twg * thg - inter_t + _EPS)

    loss_obj = jnp.sum((pcf_t - iou_t) ** 2) / _B * _OBJECT_SCALE

    scale = jnp.sqrt(2.0 - twg * thg * (1.0 / (_G * _G)))
    d0 = (sx_t - (cx - gxf)) * scale
    d1 = (sy_t - (cy - gyf)) * scale
    d2 = (tws - jnp.log(twg / awb)) * scale
    d3 = (ths - jnp.log(thg / ahb)) * scale
    loss_coords = (jnp.sum(d0 * d0) + jnp.sum(d1 * d1) + jnp.sum(d2 * d2)
                   + jnp.sum(d3 * d3)) / (_B * 4) * _COORD_SCALE

    li = jax.lax.broadcasted_iota(jnp.int32, (_R, _L), 1)
    gl = li % _GG
    gxg = (gl % _G).astype(f32)
    gyg = (gl // _G).astype(f32)

    bx1s, bx2s, by1s, by2s, a1ts, pc2s = [], [], [], [], [], []
    acc_pr_raw = jnp.float32(0.0)
    for a, (aw, ah) in enumerate(_ANCHORS):
        tx = xd_ref[a * 5 + 0]
        ty = xd_ref[a * 5 + 1]
        tw2 = xd_ref[a * 5 + 2]
        th2 = xd_ref[a * 5 + 3]
        cf = xd_ref[a * 5 + 4]
        sx = jax.nn.sigmoid(tx)
        sy = jax.nn.sigmoid(ty)
        pc = jax.nn.sigmoid(cf)
        bx = sx + gxg
        by = sy + gyg
        bw = jnp.exp(tw2) * aw
        bh = jnp.exp(th2) * ah
        bx1s.append((bx - bw * 0.5) * _ONE_T)
        bx2s.append((bx + bw * 0.5) * _ONE_T)
        by1s.append(by - bh * 0.5)
        by2s.append(by + bh * 0.5)
        a1ts.append(bw * bh * _THRESH)
        pc2s.append(pc * pc)
        acc_pr_raw = acc_pr_raw + jnp.sum(
            (sx - 0.5) ** 2 + (sy - 0.5) ** 2 + tw2 * tw2 + th2 * th2)

    BX1 = jnp.concatenate(bx1s, axis=0)
    BX2 = jnp.concatenate(bx2s, axis=0)
    BY1 = jnp.concatenate(by1s, axis=0)
    BY2 = jnp.concatenate(by2s, axis=0)
    A1T = jnp.concatenate(a1ts, axis=0)
    PC2 = jnp.concatenate(pc2s, axis=0)

    def tstep(t, over):
        cxs = ts_ref[t, 2] * _G
        cys = ts_ref[t, 3] * _G
        tws_ = ts_ref[t, 4] * _G
        ths_ = ts_ref[t, 5] * _G
        tx1 = (cxs - tws_ * 0.5) * _ONE_T
        tx2 = (cxs + tws_ * 0.5) * _ONE_T
        ty1 = cys - ths_ * 0.5
        ty2 = cys + ths_ * 0.5
        rhs = _THRESH * (tws_ * ths_ + _EPS)
        iw_ = jnp.maximum(jnp.minimum(BX2, tx2) - jnp.maximum(BX1, tx1), 0.0)
        ih_ = jnp.minimum(BY2, ty2) - jnp.maximum(BY1, ty1)
        it = iw_ * ih_
        return jnp.maximum(over, it - (A1T + rhs))

    over = jax.lax.fori_loop(0, _B, tstep, jnp.full((_A * _R, _L), -1.0, f32))
    noobj_all = (over <= 0.0).astype(f32)
    nn_raw = jnp.sum(PC2 * noobj_all)
    nc_raw = jnp.sum(noobj_all)

    cxr = tgtT_ref[2:3, :] * _G
    cyr = tgtT_ref[3:4, :] * _G
    twr = tgtT_ref[4:5, :] * _G
    thr = tgtT_ref[5:6, :] * _G
    tx1r = (cxr - twr * 0.5) * _ONE_T
    tx2r = (cxr + twr * 0.5) * _ONE_T
    ty1r = cyr - thr * 0.5
    ty2r = cyr + thr * 0.5
    rhsr = _THRESH * (twr * thr + _EPS)
    px1 = (px - pw * 0.5) * _ONE_T
    px2 = (px + pw * 0.5) * _ONE_T
    py1 = py - ph * 0.5
    py2 = py + ph * 0.5
    a1s = pw * ph * _THRESH
    iw_m = jnp.maximum(jnp.minimum(px2, tx2r) - jnp.maximum(px1, tx1r), 0.0)
    ih_m = jnp.minimum(py2, ty2r) - jnp.maximum(py1, ty1r)
    d_m = iw_m * ih_m - (a1s + rhsr)
    noobj_sel = (jnp.max(d_m, axis=1, keepdims=True) <= 0.0).astype(f32)
    nn_corr = jnp.sum(pcf_t * pcf_t * noobj_sel)
    nc_corr = jnp.sum(noobj_sel)
    pr_corr = jnp.sum((sx_t - 0.5) ** 2 + (sy_t - 0.5) ** 2
                      + tws * tws + ths * ths)

    loss_noobj = (nn_raw - nn_corr) / jnp.maximum(nc_raw - nc_corr, 1.0) \
        * _NOOBJECT_SCALE
    loss_prior = (acc_pr_raw - pr_corr) \
        / float(max((_B * _A * _GG - _B) * 4.0, 1.0)) * gate_ref[0, 0]

    total = loss_cls + loss_obj + loss_coords + loss_noobj + loss_prior
    out_ref[:, :] = jnp.full((1, 1), total, f32)


def kernel(x, targets, seen):
    xt = jnp.transpose(x.reshape(_B, _CH, _GG), (1, 0, 2))
    xd = jnp.transpose(x.reshape(_B, _A, _C + 5, _GG)[:, :, :5],
                       (1, 2, 0, 3)).reshape(_A * 5, _R, _L)
    gate = jnp.where(jnp.asarray(seen) < 12800, 0.01, 0.0)
    gate = gate.astype(jnp.float32).reshape(1, 1)
    out = pl.pallas_call(
        _body,
        out_shape=jax.ShapeDtypeStruct((1, 1), jnp.float32),
        in_specs=[
            pl.BlockSpec(memory_space=pltpu.VMEM),
            pl.BlockSpec(memory_space=pltpu.VMEM),
            pl.BlockSpec(memory_space=pltpu.VMEM),
            pl.BlockSpec(memory_space=pltpu.VMEM),
            pl.BlockSpec(memory_space=pltpu.SMEM),
            pl.BlockSpec(memory_space=pltpu.SMEM),
        ],
        out_specs=pl.BlockSpec(memory_space=pltpu.VMEM),
    )(xt, xd, targets, targets.T, targets, gate)
    return out.reshape(())
